# baseline (speedup 1.0000x reference)
import functools

import jax
import jax.numpy as jnp
from jax import lax
from jax.experimental import pallas as pl
from jax.experimental.pallas import tpu as pltpu
from jax.experimental.pallas import tpu_sc as plsc

EMB_DIM = 64
N_IDS = 4096 * 200
NUM_WORKERS = 32
ROWS_PER_W = N_IDS // NUM_WORKERS
CHUNK = 512
N_CHUNKS = ROWS_PER_W // CHUNK

_mesh = plsc.VectorSubcoreMesh(core_axis_name="c", subcore_axis_name="s")


@functools.partial(
    pl.kernel,
    mesh=_mesh,
    compiler_params=pltpu.CompilerParams(use_tc_tiling_on_sc=False),
    out_type=jax.ShapeDtypeStruct((N_IDS, EMB_DIM), jnp.float32),
    scratch_types=[
        pltpu.VMEM((CHUNK,), jnp.int32),
        pltpu.VMEM((CHUNK, EMB_DIM), jnp.float32),
        pltpu.SemaphoreType.DMA,
    ],
)
def _gather_kernel(ids_hbm, table_hbm, out_hbm, idx_v, rows_v, sem):
    wid = lax.axis_index("s") * 2 + lax.axis_index("c")
    base = wid * ROWS_PER_W

    def body(i, carry):
        off = pl.multiple_of(base + i * CHUNK, CHUNK)
        pltpu.sync_copy(ids_hbm.at[pl.ds(off, CHUNK)], idx_v)
        pltpu.async_copy(table_hbm.at[idx_v], rows_v, sem).wait()
        pltpu.sync_copy(rows_v, out_hbm.at[pl.ds(off, CHUNK)])
        return carry

    lax.fori_loop(0, N_CHUNKS, body, 0)


def kernel(input_ids, weight):
    ids_flat = input_ids.reshape(-1).astype(jnp.int32)
    out = _gather_kernel(ids_flat, weight)
    return out.reshape(input_ids.shape + (EMB_DIM,))


# speedup vs baseline: 1.0438x; 1.0438x over previous
import functools

import jax
import jax.numpy as jnp
from jax import lax
from jax.experimental import pallas as pl
from jax.experimental.pallas import tpu as pltpu
from jax.experimental.pallas import tpu_sc as plsc

EMB_DIM = 64
N_IDS = 4096 * 200
NUM_WORKERS = 32
ROWS_PER_W = N_IDS // NUM_WORKERS
CHUNK = 400
N_CHUNKS = ROWS_PER_W // CHUNK
NBUF = 4
LAG = 2
N_GROUPS = N_CHUNKS // NBUF

_mesh = plsc.VectorSubcoreMesh(core_axis_name="c", subcore_axis_name="s")


@functools.partial(
    pl.kernel,
    mesh=_mesh,
    compiler_params=pltpu.CompilerParams(use_tc_tiling_on_sc=False),
    out_type=jax.ShapeDtypeStruct((N_IDS, EMB_DIM), jnp.float32),
    scratch_types=[
        pltpu.VMEM((ROWS_PER_W,), jnp.int32),
        pltpu.VMEM((NBUF, CHUNK, EMB_DIM), jnp.float32),
        pltpu.SemaphoreType.DMA,
        pltpu.SemaphoreType.DMA,
        pltpu.SemaphoreType.DMA,
        pltpu.SemaphoreType.DMA,
        pltpu.SemaphoreType.DMA,
        pltpu.SemaphoreType.DMA,
        pltpu.SemaphoreType.DMA,
        pltpu.SemaphoreType.DMA,
    ],
)
def _gather_kernel(ids_hbm, table_hbm, out_hbm, idx_all, rows, *sems):
    gsem = sems[:NBUF]
    ssem = sems[NBUF:]
    wid = lax.axis_index("s") * 2 + lax.axis_index("c")
    base = wid * ROWS_PER_W

    def gather_copy(c, b):
        return pltpu.make_async_copy(
            table_hbm.at[idx_all.at[pl.ds(pl.multiple_of(c * CHUNK, CHUNK), CHUNK)]],
            rows.at[b],
            gsem[b],
        )

    def scatter_copy(c, b):
        return pltpu.make_async_copy(
            rows.at[b],
            out_hbm.at[pl.ds(pl.multiple_of(base + c * CHUNK, CHUNK), CHUNK)],
            ssem[b],
        )

    pltpu.sync_copy(ids_hbm.at[pl.ds(pl.multiple_of(base, ROWS_PER_W), ROWS_PER_W)],
                    idx_all)

    for b in range(LAG):
        gather_copy(b, b).start()

    def group(g, carry):
        for b in range(NBUF):
            v = g * NBUF + b
            bp = (b - LAG) % NBUF

            @pl.when(g > 0)
            def _wait_buf():
                scatter_copy(v - NBUF, b).wait()

            def _issue_gather():
                gather_copy(v, b).start()

            def _retire():
                p = v - LAG
                gather_copy(p, bp).wait()
                scatter_copy(p, bp).start()

            if b >= LAG:
                _issue_gather()
                _retire()
            else:
                pl.when(g > 0)(_issue_gather)
                pl.when(g > 0)(_retire)
        return carry

    lax.fori_loop(0, N_GROUPS, group, 0)

    for k in range(LAG):
        p = N_CHUNKS - LAG + k
        bp = p % NBUF
        gather_copy(p, bp).wait()
        scatter_copy(p, bp).start()
    for b in range(NBUF):
        p = N_CHUNKS - NBUF + b
        scatter_copy(p, b).wait()


def kernel(input_ids, weight):
    ids_flat = input_ids.reshape(-1).astype(jnp.int32)
    out = _gather_kernel(ids_flat, weight)
    return out.reshape(input_ids.shape + (EMB_DIM,))


# speedup vs baseline: 1.3859x; 1.3278x over previous
import functools

import jax
import jax.numpy as jnp
from jax import lax
from jax.experimental import pallas as pl
from jax.experimental.pallas import tpu as pltpu
from jax.experimental.pallas import tpu_sc as plsc

EMB_DIM = 64
N_IDS = 4096 * 200
NUM_WORKERS = 32
ROWS_PER_W = N_IDS // NUM_WORKERS
CHUNK = 400
N_CHUNKS = ROWS_PER_W // CHUNK
NBUF = 4
LAG = 2
N_GROUPS = N_CHUNKS // NBUF

_mesh = plsc.VectorSubcoreMesh(core_axis_name="c", subcore_axis_name="s")


@functools.partial(
    pl.kernel,
    mesh=_mesh,
    compiler_params=pltpu.CompilerParams(use_tc_tiling_on_sc=False),
    out_type=jax.ShapeDtypeStruct((N_IDS, 2 * EMB_DIM), jnp.float32),
    scratch_types=[
        pltpu.VMEM((ROWS_PER_W,), jnp.int32),
        pltpu.VMEM((NBUF, CHUNK, EMB_DIM), jnp.float32),
        pltpu.SemaphoreType.DMA,
        pltpu.SemaphoreType.DMA,
        pltpu.SemaphoreType.DMA,
        pltpu.SemaphoreType.DMA,
        pltpu.SemaphoreType.DMA,
        pltpu.SemaphoreType.DMA,
        pltpu.SemaphoreType.DMA,
        pltpu.SemaphoreType.DMA,
    ],
)
def _gather_kernel(ids_hbm, table_hbm, out_hbm, idx_all, rows, *sems):
    gsem = sems[:NBUF]
    ssem = sems[NBUF:]
    wid = lax.axis_index("s") * 2 + lax.axis_index("c")
    base = wid * ROWS_PER_W

    def gather_copy(c, b):
        return pltpu.make_async_copy(
            table_hbm.at[idx_all.at[pl.ds(pl.multiple_of(c * CHUNK, CHUNK), CHUNK)]],
            rows.at[b],
            gsem[b],
        )

    def scatter_copy(c, b):
        return pltpu.make_async_copy(
            rows.at[b],
            out_hbm.at[pl.ds(pl.multiple_of(base + c * CHUNK, CHUNK), CHUNK),
                       pl.ds(0, EMB_DIM)],
            ssem[b],
        )

    pltpu.sync_copy(ids_hbm.at[pl.ds(pl.multiple_of(base, ROWS_PER_W), ROWS_PER_W)],
                    idx_all)

    for b in range(LAG):
        gather_copy(b, b).start()

    def group(g, carry):
        for b in range(NBUF):
            v = g * NBUF + b
            bp = (b - LAG) % NBUF

            @pl.when(g > 0)
            def _wait_buf():
                scatter_copy(v - NBUF, b).wait()

            def _issue_gather():
                gather_copy(v, b).start()

            def _retire():
                p = v - LAG
                gather_copy(p, bp).wait()
                scatter_copy(p, bp).start()

            if b >= LAG:
                _issue_gather()
                _retire()
            else:
                pl.when(g > 0)(_issue_gather)
                pl.when(g > 0)(_retire)
        return carry

    lax.fori_loop(0, N_GROUPS, group, 0)

    for k in range(LAG):
        p = N_CHUNKS - LAG + k
        bp = p % NBUF
        gather_copy(p, bp).wait()
        scatter_copy(p, bp).start()
    for b in range(NBUF):
        p = N_CHUNKS - NBUF + b
        scatter_copy(p, b).wait()


def kernel(input_ids, weight):
    ids_flat = input_ids.reshape(-1).astype(jnp.int32)
    out = _gather_kernel(ids_flat, weight)
    return out.reshape(4096, 200, 2 * EMB_DIM)[:, :, :EMB_DIM]
